# baseline (speedup 1.0000x reference)
import jax
import jax.numpy as jnp
from jax import lax
from jax.experimental import pallas as pl
from jax.experimental.pallas import tpu as pltpu
from jax.experimental.pallas import tpu_sc as plsc

_MAX_LEN = 512
_D = 64
_TAB_ROWS = 2 * _MAX_LEN - 1
_TAB_WORDS = _TAB_ROWS * _D
_ROW_WORDS = _MAX_LEN * _D
_NUM_CORES = 2
_NUM_SUBCORES = 16
_NUM_WORKERS = _NUM_CORES * _NUM_SUBCORES
_WINDOW = 16


def _build_sc_kernel(batch):
    total_rows = batch * _MAX_LEN
    per_w = total_rows // _NUM_WORKERS
    mesh = plsc.VectorSubcoreMesh(core_axis_name="c", subcore_axis_name="s")

    def body(ftabs_hbm, out_hbm, sem):
        wid = lax.axis_index("c") * _NUM_SUBCORES + lax.axis_index("s")
        base = wid * per_w
        copies = []
        for t in range(per_w):
            r = base + t
            b = lax.div(r, _MAX_LEN)
            i = lax.rem(r, _MAX_LEN)
            off = _MAX_LEN - 1 - i
            s = lax.rem(off, 8)
            a = pl.multiple_of(off - s, 8)
            copies.append(
                pltpu.async_copy(ftabs_hbm.at[s, pl.ds(a, _MAX_LEN), :],
                                 out_hbm.at[b, i], sem))
            if t >= _WINDOW:
                copies[t - _WINDOW].wait()
        for t in range(per_w - _WINDOW, per_w):
            copies[t].wait()

    return pl.kernel(
        body,
        out_type=jax.ShapeDtypeStruct(
            (batch, _MAX_LEN, _MAX_LEN, _D), jnp.float32),
        mesh=mesh,
        scratch_types=[
            pltpu.SemaphoreType.DMA,
        ],
        compiler_params=pltpu.CompilerParams(use_tc_tiling_on_sc=True),
    )


def kernel(x, table):
    batch, seq_len = x.shape
    del seq_len
    ftab = jnp.flip(table, axis=0)
    ftabs = jnp.stack([ftab[s:s + 1016] for s in range(8)])
    return _build_sc_kernel(batch)(ftabs)


# speedup vs baseline: 28.9059x; 28.9059x over previous
import jax
import jax.numpy as jnp
from jax import lax
from jax.experimental import pallas as pl

_MAX_LEN = 512
_D = 64
_PHASE_LEN = 1016
_BI = 8


def _tc_body(ftabs_ref, out_ref):
    ib = pl.program_id(1)
    i0 = ib * _BI
    for u in range(_BI):
        s = (_MAX_LEN - 1 - u) % 8
        c = _MAX_LEN - 1 - u - s
        a = pl.multiple_of(c - i0, 8)
        out_ref[0, u] = ftabs_ref[s, pl.ds(a, _MAX_LEN), :]


def _build_tc_kernel(batch):
    return pl.pallas_call(
        _tc_body,
        grid=(batch, _MAX_LEN // _BI),
        in_specs=[pl.BlockSpec((8, _PHASE_LEN, _D), lambda b, ib: (0, 0, 0))],
        out_specs=pl.BlockSpec((1, _BI, _MAX_LEN, _D),
                               lambda b, ib: (b, ib, 0, 0)),
        out_shape=jax.ShapeDtypeStruct(
            (batch, _MAX_LEN, _MAX_LEN, _D), jnp.float32),
    )


def kernel(x, table):
    batch, seq_len = x.shape
    del seq_len
    ftab = jnp.flip(table, axis=0)
    ftabs = jnp.stack([ftab[s:s + _PHASE_LEN] for s in range(8)])
    return _build_tc_kernel(batch)(ftabs)


# speedup vs baseline: 29.6769x; 1.0267x over previous
import jax
import jax.numpy as jnp
from jax import lax
from jax.experimental import pallas as pl
from jax.experimental.pallas import tpu as pltpu
from jax.experimental.pallas import tpu_sc as plsc

_MAX_LEN = 512
_D = 64
_TAB_ROWS = 2 * _MAX_LEN - 1
_TAB_WORDS = _TAB_ROWS * _D
_ROW_WORDS = _MAX_LEN * _D
_NUM_CORES = 2
_NUM_SUBCORES = 16
_NUM_WORKERS = _NUM_CORES * _NUM_SUBCORES
_WINDOW = 16


def _build_sc_kernel(batch):
    total_rows = batch * _MAX_LEN
    per_w = total_rows // _NUM_WORKERS
    mesh = plsc.VectorSubcoreMesh(core_axis_name="c", subcore_axis_name="s")

    def body(ftab_hbm, out_hbm, tab_v, sem):
        wid = lax.axis_index("c") * _NUM_SUBCORES + lax.axis_index("s")
        base = wid * per_w
        pltpu.sync_copy(ftab_hbm, tab_v)
        copies = []
        for t in range(per_w):
            r = base + t
            b = lax.div(r, _MAX_LEN)
            i = lax.rem(r, _MAX_LEN)
            off = _MAX_LEN - 1 - i
            copies.append(
                pltpu.async_copy(tab_v.at[pl.ds(off, _MAX_LEN), :],
                                 out_hbm.at[b, i], sem))
            if t >= _WINDOW:
                copies[t - _WINDOW].wait()
        for t in range(per_w - _WINDOW, per_w):
            copies[t].wait()

    return pl.kernel(
        body,
        out_type=jax.ShapeDtypeStruct(
            (batch, _MAX_LEN, _MAX_LEN, _D), jnp.float32),
        mesh=mesh,
        scratch_types=[
            pltpu.VMEM((_TAB_ROWS, _D), jnp.float32),
            pltpu.SemaphoreType.DMA,
        ],
    )


def kernel(x, table):
    batch, seq_len = x.shape
    del seq_len
    ftab = jnp.flip(table, axis=0)
    return _build_sc_kernel(batch)(ftab)


# speedup vs baseline: 30.4869x; 1.0273x over previous
import jax
import jax.numpy as jnp
from jax import lax
from jax.experimental import pallas as pl
from jax.experimental.pallas import tpu as pltpu
from jax.experimental.pallas import tpu_sc as plsc

_MAX_LEN = 512
_D = 64
_PK = 511
_NUM_CORES = 2
_NUM_SUBCORES = 16
_NUM_WORKERS = _NUM_CORES * _NUM_SUBCORES
_WINDOW = 16


def _build_sc_kernel(batch):
    total_rows = batch * _MAX_LEN
    per_w = total_rows // _NUM_WORKERS
    mesh = plsc.VectorSubcoreMesh(core_axis_name="c", subcore_axis_name="s")

    def body(ctab_hbm, out_hbm, tab_v, sem):
        wid = lax.axis_index("c") * _NUM_SUBCORES + lax.axis_index("s")
        base = wid * per_w
        pltpu.sync_copy(ctab_hbm, tab_v)
        copies = []
        for t in range(per_w):
            r = base + t
            b = lax.div(r, _MAX_LEN)
            i = lax.rem(r, _MAX_LEN)
            off = _MAX_LEN - 1 - i
            par = (_MAX_LEN - 1 - t) % 2
            k = lax.shift_right_logical(off, 1)
            copies.append(
                pltpu.async_copy(tab_v.at[pl.ds(par * _PK + k, _MAX_LEN // 2), :],
                                 out_hbm.at[b, i], sem))
            if t >= _WINDOW:
                copies[t - _WINDOW].wait()
        for t in range(per_w - _WINDOW, per_w):
            copies[t].wait()

    return pl.kernel(
        body,
        out_type=jax.ShapeDtypeStruct(
            (batch, _MAX_LEN, _MAX_LEN // 2, 2 * _D), jnp.float32),
        mesh=mesh,
        scratch_types=[
            pltpu.VMEM((2 * _PK, 2 * _D), jnp.float32),
            pltpu.SemaphoreType.DMA,
        ],
    )


def kernel(x, table):
    batch, seq_len = x.shape
    ftab = jnp.flip(table, axis=0)
    flat = jnp.concatenate(
        [ftab.reshape(-1), jnp.zeros(3 * _D, jnp.float32)])
    even = flat[: _PK * 2 * _D].reshape(_PK, 2 * _D)
    odd = flat[_D: _D + _PK * 2 * _D].reshape(_PK, 2 * _D)
    ctab = jnp.concatenate([even, odd])
    out = _build_sc_kernel(batch)(ctab)
    return out.reshape(batch, seq_len, seq_len, _D)
